# baseline (speedup 1.0000x reference)
import jax
import jax.numpy as jnp
from jax.experimental import pallas as pl
from jax.experimental.pallas import tpu as pltpu


def _body(m_ref, x_ref, o_ref):
    g, h, w = x_ref.shape
    mg = m_ref.shape[0]
    x = x_ref[...].reshape(mg, g // mg, h, w)
    o_ref[...] = (x * m_ref[...][:, None]).reshape(g, h, w)


def kernel(input, mask):
    B, C, K, H, W = input.shape
    BC = B * C
    x = input.reshape(BC * K, H, W)
    m = mask.reshape(BC, H, W)

    ROWS = 64
    n = (BC * K) // ROWS

    out = pl.pallas_call(
        _body,
        grid=(n,),
        in_specs=[
            pl.BlockSpec((ROWS // K, H, W), lambda j: (j, 0, 0)),
            pl.BlockSpec((ROWS, H, W), lambda j: (j, 0, 0)),
        ],
        out_specs=pl.BlockSpec((ROWS, H, W), lambda j: (j, 0, 0)),
        out_shape=jax.ShapeDtypeStruct((BC * K, H, W), x.dtype),
        compiler_params=pltpu.CompilerParams(
            dimension_semantics=("parallel",),
            vmem_limit_bytes=110 * 1024 * 1024,
        ),
    )(m, x)
    return out.reshape(B, C, K, H, W)


# speedup vs baseline: 1.0017x; 1.0017x over previous
import jax
from jax.experimental import pallas as pl
from jax.experimental.pallas import tpu as pltpu


def _body(m_ref, x_ref, o_ref):
    g, h, w = x_ref.shape
    mg = m_ref.shape[0]
    x = x_ref[...].reshape(mg, g // mg, h, w)
    o_ref[...] = (x * m_ref[...][:, None]).reshape(g, h, w)


def kernel(input, mask):
    B, C, K, H, W = input.shape
    BC = B * C
    x = input.reshape(BC * K, H, W)
    m = mask.reshape(BC, H, W)

    ROWS = 64
    n = (BC * K) // ROWS

    out = pl.pallas_call(
        _body,
        grid=(n,),
        in_specs=[
            pl.BlockSpec((ROWS // K, H, W), lambda j: (j, 0, 0)),
            pl.BlockSpec((ROWS, H, W), lambda j: (j, 0, 0)),
        ],
        out_specs=pl.BlockSpec((ROWS, H, W), lambda j: (j, 0, 0)),
        out_shape=jax.ShapeDtypeStruct((BC * K, H, W), x.dtype),
        compiler_params=pltpu.CompilerParams(
            dimension_semantics=("arbitrary",),
            vmem_limit_bytes=62 * 1024 * 1024,
        ),
    )(m, x)
    return out.reshape(B, C, K, H, W)
